# baseline (speedup 1.0000x reference)
import functools

import jax
import jax.numpy as jnp
from jax import lax
from jax.experimental import pallas as pl
from jax.experimental.pallas import tpu as pltpu
from jax.experimental.pallas import tpu_sc as plsc

NUM_EXPERTS = 64
TOP_K = 8
HIDDEN = 1024
ROW_TILE = 4096
N_ROWS = 32768

NC = 2
NS = 16
NW = NC * NS
ROWS_PER_W = N_ROWS // NW
GROUPS = ROWS_PER_W // 16


def _matmul_body(hs_ref, w_ref, logits_ref):
    hs_bf = hs_ref[...].astype(jnp.bfloat16)
    w_bf = w_ref[...].astype(jnp.bfloat16)
    logits_ref[...] = jax.lax.dot_general(
        hs_bf, w_bf,
        dimension_numbers=(((1,), (1,)), ((), ())),
        preferred_element_type=jnp.float32,
    )


def _sc_topk_kernel(logits_flat_hbm, topv_hbm, topi_hbm, lg_v, tv_v, ti_v):
    wid = lax.axis_index("s") * NC + lax.axis_index("c")
    base = wid * ROWS_PER_W
    pltpu.sync_copy(
        logits_flat_hbm.at[pl.ds(base * NUM_EXPERTS, ROWS_PER_W * NUM_EXPERTS)],
        lg_v)

    lane = lax.broadcasted_iota(jnp.int32, (16,), 0)
    col = lane * NUM_EXPERTS
    out_col = lane * TOP_K
    neg_inf = jnp.full((16,), -jnp.inf, dtype=jnp.float32)
    zero_i = jnp.zeros((16,), dtype=jnp.int32)

    def group_body(g, _):
        gbase = g * (16 * NUM_EXPERTS)
        t = [neg_inf] * TOP_K
        ti = [zero_i] * TOP_K
        for e in range(NUM_EXPERTS):
            idx = col + (gbase + e)
            v = plsc.load_gather(lg_v, [idx])
            ev = jnp.full((16,), e, dtype=jnp.int32)
            c = [v > t[j] for j in range(TOP_K)]
            nt = []
            nti = []
            for j in range(TOP_K):
                if j == 0:
                    nt.append(jnp.where(c[0], v, t[0]))
                    nti.append(jnp.where(c[0], ev, ti[0]))
                else:
                    nt.append(jnp.where(c[j - 1], t[j - 1],
                                        jnp.where(c[j], v, t[j])))
                    nti.append(jnp.where(c[j - 1], ti[j - 1],
                                         jnp.where(c[j], ev, ti[j])))
            t = nt
            ti = nti
        m = t[0]
        es = [jnp.exp(t[j] - m) for j in range(TOP_K)]
        s = es[0]
        for j in range(1, TOP_K):
            s = s + es[j]
        obase = g * (16 * TOP_K)
        for j in range(TOP_K):
            oidx = out_col + (obase + j)
            plsc.store_scatter(tv_v, [oidx], es[j] / s)
            plsc.store_scatter(ti_v, [oidx], ti[j])
        return _

    lax.fori_loop(0, GROUPS, group_body, None)

    pltpu.sync_copy(tv_v, topv_hbm.at[pl.ds(base * TOP_K, ROWS_PER_W * TOP_K)])
    pltpu.sync_copy(ti_v, topi_hbm.at[pl.ds(base * TOP_K, ROWS_PER_W * TOP_K)])


def _sc_topk(logits):
    mesh = plsc.VectorSubcoreMesh(core_axis_name="c", subcore_axis_name="s")
    fn = functools.partial(
        pl.kernel,
        mesh=mesh,
        out_type=[
            jax.ShapeDtypeStruct((N_ROWS * TOP_K,), jnp.float32),
            jax.ShapeDtypeStruct((N_ROWS * TOP_K,), jnp.int32),
        ],
        scratch_types=[
            pltpu.VMEM((ROWS_PER_W * NUM_EXPERTS,), jnp.float32),
            pltpu.VMEM((ROWS_PER_W * TOP_K,), jnp.float32),
            pltpu.VMEM((ROWS_PER_W * TOP_K,), jnp.int32),
        ],
        compiler_params=pltpu.CompilerParams(needs_layout_passes=False),
    )(_sc_topk_kernel)
    topv_flat, topi_flat = fn(logits.reshape(-1))
    return (topv_flat.reshape(N_ROWS, TOP_K), topi_flat.reshape(N_ROWS, TOP_K))


def kernel(hidden_states, weight):
    hs = hidden_states.reshape(-1, HIDDEN)
    logits = pl.pallas_call(
        _matmul_body,
        grid=(N_ROWS // ROW_TILE,),
        in_specs=[
            pl.BlockSpec((ROW_TILE, HIDDEN), lambda i: (i, 0)),
            pl.BlockSpec((NUM_EXPERTS, HIDDEN), lambda i: (0, 0)),
        ],
        out_specs=pl.BlockSpec((ROW_TILE, NUM_EXPERTS), lambda i: (i, 0)),
        out_shape=jax.ShapeDtypeStruct((N_ROWS, NUM_EXPERTS), jnp.float32),
        compiler_params=pltpu.CompilerParams(
            dimension_semantics=("arbitrary",),
        ),
    )(hs, weight)
    topv, topi = _sc_topk(logits)
    return (logits, topv, topi)


# speedup vs baseline: 1.5283x; 1.5283x over previous
import functools

import jax
import jax.numpy as jnp
from jax import lax
from jax.experimental import pallas as pl
from jax.experimental.pallas import tpu as pltpu

NUM_EXPERTS = 64
TOP_K = 8
HIDDEN = 1024
ROW_TILE = 4096
K_SPLIT = 2
K_CHUNK = HIDDEN // K_SPLIT


def _router_body(hs_ref, w_ref, logits_ref, topv_ref, topi_ref, lt_acc):
    j = pl.program_id(1)
    hs_bf = hs_ref[...].astype(jnp.bfloat16)
    w_bf = w_ref[...].astype(jnp.bfloat16)
    part = jax.lax.dot_general(
        hs_bf, w_bf,
        dimension_numbers=(((1,), (1,)), ((), ())),
        preferred_element_type=jnp.float32,
    )
    part_t = jax.lax.dot_general(
        w_bf, hs_bf,
        dimension_numbers=(((1,), (1,)), ((), ())),
        preferred_element_type=jnp.float32,
    )

    @pl.when(j == 0)
    def _():
        logits_ref[...] = part
        lt_acc[...] = part_t

    @pl.when(j != 0)
    def _():
        logits_ref[...] += part
        lt_acc[...] += part_t

    @pl.when(j == K_SPLIT - 1)
    def _():
        r = ROW_TILE
        iota = lax.broadcasted_iota(jnp.int32, (NUM_EXPERTS, r), 0)
        work = lt_acc[...]
        vals = []
        idxs = []
        for _k in range(TOP_K):
            m = jnp.max(work, axis=0, keepdims=True)
            is_max = work == m
            idx = jnp.min(jnp.where(is_max, iota, NUM_EXPERTS), axis=0,
                          keepdims=True)
            vals.append(m)
            idxs.append(idx)
            work = jnp.where(iota == idx, -jnp.inf, work)
        topl = jnp.concatenate(vals, axis=0)
        topi = jnp.concatenate(idxs, axis=0)

        e = jnp.exp(topl - topl[0:1, :])
        topv = e / jnp.sum(e, axis=0, keepdims=True)
        topv_ref[...] = topv.T
        topi_ref[...] = topi.T


def kernel(hidden_states, weight, interpret=False):
    hs = hidden_states.reshape(-1, HIDDEN)
    n_rows = hs.shape[0]
    grid = (n_rows // ROW_TILE, K_SPLIT)
    logits, topv, topi = pl.pallas_call(
        _router_body,
        grid=grid,
        in_specs=[
            pl.BlockSpec((ROW_TILE, K_CHUNK), lambda i, j: (i, j)),
            pl.BlockSpec((NUM_EXPERTS, K_CHUNK), lambda i, j: (0, j)),
        ],
        out_specs=[
            pl.BlockSpec((ROW_TILE, NUM_EXPERTS), lambda i, j: (i, 0)),
            pl.BlockSpec((ROW_TILE, TOP_K), lambda i, j: (i, 0)),
            pl.BlockSpec((ROW_TILE, TOP_K), lambda i, j: (i, 0)),
        ],
        out_shape=[
            jax.ShapeDtypeStruct((n_rows, NUM_EXPERTS), jnp.float32),
            jax.ShapeDtypeStruct((n_rows, TOP_K), jnp.float32),
            jax.ShapeDtypeStruct((n_rows, TOP_K), jnp.int32),
        ],
        scratch_shapes=[pltpu.VMEM((NUM_EXPERTS, ROW_TILE), jnp.float32)],
        compiler_params=pltpu.CompilerParams(
            dimension_semantics=("arbitrary", "arbitrary"),
        ),
        interpret=interpret,
    )(hs, weight)
    return (logits, topv, topi)


# speedup vs baseline: 2.0197x; 1.3216x over previous
import functools

import jax
import jax.numpy as jnp
from jax import lax
from jax.experimental import pallas as pl
from jax.experimental.pallas import tpu as pltpu

NUM_EXPERTS = 64
TOP_K = 8
HIDDEN = 1024
ROW_TILE = 2048


def _router_body(hs_ref, w_ref, logits_ref, topv_ref, topi_ref):
    hs = hs_ref[...]
    w = w_ref[...]
    hs_bf = hs.astype(jnp.bfloat16)
    w_bf = w.astype(jnp.bfloat16)
    logits = jax.lax.dot_general(
        hs_bf, w_bf,
        dimension_numbers=(((1,), (1,)), ((), ())),
        preferred_element_type=jnp.float32,
    )
    logits_ref[...] = logits

    logits_t = jax.lax.dot_general(
        w_bf, hs_bf,
        dimension_numbers=(((1,), (1,)), ((), ())),
        preferred_element_type=jnp.float32,
    )

    r = logits.shape[0]
    iota = lax.broadcasted_iota(jnp.int32, (NUM_EXPERTS, r), 0)
    work = logits_t
    vals = []
    idxs = []
    for _ in range(TOP_K):
        m = jnp.max(work, axis=0, keepdims=True)
        is_max = work == m
        idx = jnp.min(jnp.where(is_max, iota, NUM_EXPERTS), axis=0,
                      keepdims=True)
        vals.append(m)
        idxs.append(idx)
        work = jnp.where(iota == idx, -jnp.inf, work)
    topl = jnp.concatenate(vals, axis=0)
    topi = jnp.concatenate(idxs, axis=0)

    e = jnp.exp(topl - topl[0:1, :])
    topv = e / jnp.sum(e, axis=0, keepdims=True)
    topv_ref[...] = topv.T
    topi_ref[...] = topi.T


def kernel(hidden_states, weight, interpret=False):
    hs = hidden_states.reshape(-1, HIDDEN)
    n_rows = hs.shape[0]
    grid = (n_rows // ROW_TILE,)
    logits, topv, topi = pl.pallas_call(
        _router_body,
        grid=grid,
        in_specs=[
            pl.BlockSpec((ROW_TILE, HIDDEN), lambda i: (i, 0)),
            pl.BlockSpec((NUM_EXPERTS, HIDDEN), lambda i: (0, 0)),
        ],
        out_specs=[
            pl.BlockSpec((ROW_TILE, NUM_EXPERTS), lambda i: (i, 0)),
            pl.BlockSpec((ROW_TILE, TOP_K), lambda i: (i, 0)),
            pl.BlockSpec((ROW_TILE, TOP_K), lambda i: (i, 0)),
        ],
        out_shape=[
            jax.ShapeDtypeStruct((n_rows, NUM_EXPERTS), jnp.float32),
            jax.ShapeDtypeStruct((n_rows, TOP_K), jnp.float32),
            jax.ShapeDtypeStruct((n_rows, TOP_K), jnp.int32),
        ],
        compiler_params=pltpu.CompilerParams(
            dimension_semantics=("arbitrary",),
        ),
        interpret=interpret,
    )(hs, weight)
    return (logits, topv, topi)


# speedup vs baseline: 2.0416x; 1.0108x over previous
import functools

import jax
import jax.numpy as jnp
from jax import lax
from jax.experimental import pallas as pl
from jax.experimental.pallas import tpu as pltpu

NUM_EXPERTS = 64
TOP_K = 8
HIDDEN = 1024
ROW_TILE = 2048


def _router_body(hs_ref, w_ref, logits_ref, topv_ref, topi_ref):
    hs = hs_ref[...]
    w = w_ref[...]
    logits = jax.lax.dot_general(
        hs, w,
        dimension_numbers=(((1,), (1,)), ((), ())),
        preferred_element_type=jnp.float32,
    )
    logits_ref[...] = logits

    logits_t = jax.lax.dot_general(
        w, hs,
        dimension_numbers=(((1,), (1,)), ((), ())),
        preferred_element_type=jnp.float32,
    )

    r = logits.shape[0]
    iota = lax.broadcasted_iota(jnp.int32, (NUM_EXPERTS, r), 0)
    work = logits_t
    vals = []
    idxs = []
    for _ in range(TOP_K):
        m = jnp.max(work, axis=0, keepdims=True)
        is_max = work == m
        idx = jnp.min(jnp.where(is_max, iota, NUM_EXPERTS), axis=0,
                      keepdims=True)
        vals.append(m)
        idxs.append(idx)
        work = jnp.where(iota == idx, -jnp.inf, work)
    topl = jnp.concatenate(vals, axis=0)
    topi = jnp.concatenate(idxs, axis=0)

    e = jnp.exp(topl - topl[0:1, :])
    topv = e / jnp.sum(e, axis=0, keepdims=True)
    topv_ref[...] = topv.T
    topi_ref[...] = topi.T


def kernel(hidden_states, weight, interpret=False):
    hs = hidden_states.reshape(-1, HIDDEN)
    n_rows = hs.shape[0]
    grid = (n_rows // ROW_TILE,)
    logits, topv, topi = pl.pallas_call(
        _router_body,
        grid=grid,
        in_specs=[
            pl.BlockSpec((ROW_TILE, HIDDEN), lambda i: (i, 0)),
            pl.BlockSpec((NUM_EXPERTS, HIDDEN), lambda i: (0, 0)),
        ],
        out_specs=[
            pl.BlockSpec((ROW_TILE, NUM_EXPERTS), lambda i: (i, 0)),
            pl.BlockSpec((ROW_TILE, TOP_K), lambda i: (i, 0)),
            pl.BlockSpec((ROW_TILE, TOP_K), lambda i: (i, 0)),
        ],
        out_shape=[
            jax.ShapeDtypeStruct((n_rows, NUM_EXPERTS), jnp.float32),
            jax.ShapeDtypeStruct((n_rows, TOP_K), jnp.float32),
            jax.ShapeDtypeStruct((n_rows, TOP_K), jnp.int32),
        ],
        compiler_params=pltpu.CompilerParams(
            dimension_semantics=("arbitrary",),
        ),
        interpret=interpret,
    )(hs, weight)
    return (logits, topv, topi)
